# Initial kernel scaffold; baseline (speedup 1.0000x reference)
import functools

import jax
import jax.numpy as jnp
from jax import lax
from jax.experimental import pallas as pl
from jax.experimental.pallas import tpu as pltpu
from jax.experimental.pallas import tpu_sc as plsc

N = 10000
NP = 10240
E = 320000
D = 128
NC = 2
NS = 16
NW = NC * NS
EPW = 10240
EP = NW * EPW
CH = 128
NCH = EPW // CH
RPW = NP // NS
HW = 16

_mesh = plsc.VectorSubcoreMesh(
    core_axis_name="c", subcore_axis_name="s", num_cores=NC, num_subcores=NS
)


def _hist_body(src_hbm, dst_hbm, z16_hbm, ones_hbm, out_hbm,
               acc_s, acc_d, six, dix, ones_v, sem):
    c = lax.axis_index("c")
    s = lax.axis_index("s")
    wid = c * NS + s
    pltpu.sync_copy(z16_hbm.at[pl.ds(s * RPW, RPW)], acc_s.at[pl.ds(s * RPW, RPW)])
    pltpu.sync_copy(z16_hbm.at[pl.ds(s * RPW, RPW)], acc_d.at[pl.ds(s * RPW, RPW)])
    pltpu.sync_copy(ones_hbm, ones_v)
    plsc.subcore_barrier()

    ebase = wid * EPW

    def body(i, carry):
        base = ebase + i * CH
        pltpu.sync_copy(src_hbm.at[pl.ds(base, CH)], six)
        pltpu.sync_copy(dst_hbm.at[pl.ds(base, CH)], dix)
        pltpu.sync_copy(ones_v, acc_s.at[six], add=True)
        pltpu.sync_copy(ones_v, acc_d.at[dix], add=True)
        return carry

    lax.fori_loop(0, NCH, body, 0)
    plsc.subcore_barrier()
    pltpu.sync_copy(acc_s.at[pl.ds(s * RPW, RPW)],
                    out_hbm.at[pl.ds(c * NP + s * RPW, RPW)])
    pltpu.sync_copy(acc_d.at[pl.ds(s * RPW, RPW)],
                    out_hbm.at[pl.ds(2 * NP + c * NP + s * RPW, RPW)])


_hist = pl.kernel(
    _hist_body,
    out_type=jax.ShapeDtypeStruct((4 * NP, HW), jnp.float32),
    mesh=_mesh,
    scratch_types=[
        pltpu.VMEM_SHARED((NP, HW), jnp.float32),
        pltpu.VMEM_SHARED((NP, HW), jnp.float32),
        pltpu.VMEM((CH,), jnp.int32),
        pltpu.VMEM((CH,), jnp.int32),
        pltpu.VMEM((CH, HW), jnp.float32),
        pltpu.SemaphoreType.DMA,
    ],
)


def _agg_body(h_hbm, src_hbm, dst_hbm, zeros_hbm, out_hbm,
              acc, six, dix, rows, sem):
    c = lax.axis_index("c")
    s = lax.axis_index("s")
    wid = c * NS + s
    pltpu.sync_copy(zeros_hbm.at[pl.ds(s * RPW, RPW)], acc.at[pl.ds(s * RPW, RPW)])
    plsc.subcore_barrier()

    ebase = wid * EPW

    def body(i, carry):
        base = ebase + i * CH
        pltpu.sync_copy(src_hbm.at[pl.ds(base, CH)], six)
        pltpu.sync_copy(dst_hbm.at[pl.ds(base, CH)], dix)
        pltpu.async_copy(h_hbm.at[six], rows, sem).wait()
        pltpu.sync_copy(rows, acc.at[dix], add=True)
        return carry

    lax.fori_loop(0, NCH, body, 0)
    plsc.subcore_barrier()
    pltpu.sync_copy(acc.at[pl.ds(s * RPW, RPW)],
                    out_hbm.at[pl.ds(c * NP + s * RPW, RPW)])


_agg = pl.kernel(
    _agg_body,
    out_type=jax.ShapeDtypeStruct((2 * NP, D), jnp.float32),
    mesh=_mesh,
    scratch_types=[
        pltpu.VMEM_SHARED((NP, D), jnp.float32),
        pltpu.VMEM((CH,), jnp.int32),
        pltpu.VMEM((CH,), jnp.int32),
        pltpu.VMEM((CH, D), jnp.float32),
        pltpu.SemaphoreType.DMA,
    ],
)


BR = 1280
_GRID = NP // BR


def _norm_from(h0, h1):
    deg = h0[:, :1] + h1[:, :1]
    return lax.rsqrt(jnp.maximum(deg, 1.0))


def _scale_body(x_ref, hs0, hs1, o_ref):
    o_ref[...] = x_ref[...] * _norm_from(hs0[...], hs1[...])


def _layer1_body(a0, a1, hd0, hd1, hs0, hs1, w_ref, b_ref, o_ref):
    agg = (a0[...] + a1[...]) * _norm_from(hd0[...], hd1[...])
    h = jnp.dot(agg, w_ref[...], preferred_element_type=jnp.float32) + b_ref[...]
    h = jnp.maximum(h, 0.0)
    o_ref[...] = h * _norm_from(hs0[...], hs1[...])


def _layer2_body(a0, a1, hd0, hd1, w_ref, b_ref, wf_ref, bf_ref, o_ref):
    agg = (a0[...] + a1[...]) * _norm_from(hd0[...], hd1[...])
    h = jnp.dot(agg, w_ref[...], preferred_element_type=jnp.float32) + b_ref[...]
    o_ref[...] = (
        jnp.dot(h, wf_ref[...], preferred_element_type=jnp.float32) + bf_ref[...]
    )


def _hist_spec(kind, core):
    off = (kind * 2 + core) * (NP // BR)
    return pl.BlockSpec((BR, HW), lambda i, o=off: (o + i, 0))


def _acc_spec(core):
    off = core * (NP // BR)
    return pl.BlockSpec((BR, D), lambda i, o=off: (o + i, 0))


_row_spec = pl.BlockSpec((BR, D), lambda i: (i, 0))


_scale = pl.pallas_call(
    _scale_body,
    grid=(_GRID,),
    in_specs=[_row_spec, _hist_spec(0, 0), _hist_spec(0, 1)],
    out_specs=_row_spec,
    out_shape=jax.ShapeDtypeStruct((NP, D), jnp.float32),
)

_layer1 = pl.pallas_call(
    _layer1_body,
    grid=(_GRID,),
    in_specs=[
        _acc_spec(0), _acc_spec(1),
        _hist_spec(1, 0), _hist_spec(1, 1),
        _hist_spec(0, 0), _hist_spec(0, 1),
        pl.BlockSpec((D, D), lambda i: (0, 0)),
        pl.BlockSpec((1, D), lambda i: (0, 0)),
    ],
    out_specs=_row_spec,
    out_shape=jax.ShapeDtypeStruct((NP, D), jnp.float32),
)

_layer2 = pl.pallas_call(
    _layer2_body,
    grid=(_GRID,),
    in_specs=[
        _acc_spec(0), _acc_spec(1),
        _hist_spec(1, 0), _hist_spec(1, 1),
        pl.BlockSpec((D, D), lambda i: (0, 0)),
        pl.BlockSpec((1, D), lambda i: (0, 0)),
        pl.BlockSpec((D, 1), lambda i: (0, 0)),
        pl.BlockSpec((1, 1), lambda i: (0, 0)),
    ],
    out_specs=pl.BlockSpec((BR, 1), lambda i: (i, 0)),
    out_shape=jax.ShapeDtypeStruct((NP, 1), jnp.float32),
)


def kernel(x, edge_index, W1, b1, W2, b2, Wfc, bfc):
    src = edge_index[0].astype(jnp.int32)
    dst = edge_index[1].astype(jnp.int32)
    pad = jnp.full((EP - E,), N, dtype=jnp.int32)
    src = jnp.concatenate([src, pad])
    dst = jnp.concatenate([dst, pad])
    xp = jnp.pad(x.astype(jnp.float32), ((0, NP - N), (0, 0)))

    zeros = jnp.zeros((NP, D), jnp.float32)
    z16 = jnp.zeros((NP, HW), jnp.float32)
    ones = jnp.ones((CH, HW), jnp.float32)

    hist = _hist(src, dst, z16, ones)
    h0 = _scale(xp, hist, hist)

    acc1 = _agg(h0, src, dst, zeros)
    h1 = _layer1(acc1, acc1, hist, hist, hist, hist,
                 W1, b1.reshape(1, D))

    acc2 = _agg(h1, src, dst, zeros)
    out = _layer2(acc2, acc2, hist, hist,
                  W2, b2.reshape(1, D), Wfc, bfc.reshape(1, 1))
    return out[:N]


# speedup vs baseline: 2.8615x; 2.8615x over previous
import functools

import jax
import jax.numpy as jnp
from jax import lax
from jax.experimental import pallas as pl
from jax.experimental.pallas import tpu as pltpu
from jax.experimental.pallas import tpu_sc as plsc

N = 10000
NP = 10240
E = 320000
D = 128
NC = 2
NS = 16
NW = NC * NS
EPW = 10240
EP = NW * EPW
CH = 128
NCH = EPW // CH
RPW = NP // NS
HW = 16

_mesh = plsc.VectorSubcoreMesh(
    core_axis_name="c", subcore_axis_name="s", num_cores=NC, num_subcores=NS
)


def _hist_body(src_hbm, dst_hbm, zeros_hbm, out_hbm,
               acc, six, dix, pat_s, pat_d):
    c = lax.axis_index("c")
    s = lax.axis_index("s")
    wid = c * NS + s
    one16 = jnp.ones((16,), jnp.float32)
    zero16 = jnp.zeros((16,), jnp.float32)

    def fill(i, carry):
        pat_s[i, pl.ds(0, 16)] = one16
        pat_d[i, pl.ds(0, 16)] = zero16
        pat_s[i, pl.ds(16, 16)] = zero16
        pat_d[i, pl.ds(16, 16)] = one16
        for j in range(2, 8):
            pat_s[i, pl.ds(j * 16, 16)] = zero16
            pat_d[i, pl.ds(j * 16, 16)] = zero16
        return carry

    lax.fori_loop(0, CH, fill, 0)
    pltpu.sync_copy(zeros_hbm.at[pl.ds(s * RPW, RPW)], acc.at[pl.ds(s * RPW, RPW)])
    plsc.subcore_barrier()

    ebase = wid * EPW

    def body(i, carry):
        base = ebase + i * CH
        pltpu.sync_copy(src_hbm.at[pl.ds(base, CH)], six)
        pltpu.sync_copy(dst_hbm.at[pl.ds(base, CH)], dix)
        pltpu.sync_copy(pat_s, acc.at[six], add=True)
        pltpu.sync_copy(pat_d, acc.at[dix], add=True)
        return carry

    lax.fori_loop(0, NCH, body, 0)
    plsc.subcore_barrier()
    pltpu.sync_copy(acc.at[pl.ds(s * RPW, RPW)],
                    out_hbm.at[pl.ds(c * NP + s * RPW, RPW)])


_hist = pl.kernel(
    _hist_body,
    out_type=jax.ShapeDtypeStruct((2 * NP, D), jnp.float32),
    mesh=_mesh,
    scratch_types=[
        pltpu.VMEM_SHARED((NP, D), jnp.float32),
        pltpu.VMEM((CH,), jnp.int32),
        pltpu.VMEM((CH,), jnp.int32),
        pltpu.VMEM((CH, D), jnp.float32),
        pltpu.VMEM((CH, D), jnp.float32),
    ],
)


def _agg_body(h_hbm, src_hbm, dst_hbm, zeros_hbm, out_hbm,
              acc, six, dix, rows, sem):
    c = lax.axis_index("c")
    s = lax.axis_index("s")
    wid = c * NS + s
    pltpu.sync_copy(zeros_hbm.at[pl.ds(s * RPW, RPW)], acc.at[pl.ds(s * RPW, RPW)])
    plsc.subcore_barrier()

    ebase = wid * EPW

    def body(i, carry):
        base = ebase + i * CH
        pltpu.sync_copy(src_hbm.at[pl.ds(base, CH)], six)
        pltpu.sync_copy(dst_hbm.at[pl.ds(base, CH)], dix)
        pltpu.async_copy(h_hbm.at[six], rows, sem).wait()
        pltpu.sync_copy(rows, acc.at[dix], add=True)
        return carry

    lax.fori_loop(0, NCH, body, 0)
    plsc.subcore_barrier()
    pltpu.sync_copy(acc.at[pl.ds(s * RPW, RPW)],
                    out_hbm.at[pl.ds(c * NP + s * RPW, RPW)])


_agg = pl.kernel(
    _agg_body,
    out_type=jax.ShapeDtypeStruct((2 * NP, D), jnp.float32),
    mesh=_mesh,
    scratch_types=[
        pltpu.VMEM_SHARED((NP, D), jnp.float32),
        pltpu.VMEM((CH,), jnp.int32),
        pltpu.VMEM((CH,), jnp.int32),
        pltpu.VMEM((CH, D), jnp.float32),
        pltpu.SemaphoreType.DMA,
    ],
)


BR = 1280
_GRID = NP // BR


SRC_COL = 0
DST_COL = 16


def _norm_from(h0, h1, col):
    deg = h0[:, col:col + 1] + h1[:, col:col + 1]
    return lax.rsqrt(jnp.maximum(deg, 1.0))


def _scale_body(x_ref, h0, h1, o_ref):
    o_ref[...] = x_ref[...] * _norm_from(h0[...], h1[...], SRC_COL)


def _layer1_body(a0, a1, h0, h1, w_ref, b_ref, o_ref):
    agg = (a0[...] + a1[...]) * _norm_from(h0[...], h1[...], DST_COL)
    h = jnp.dot(agg, w_ref[...], preferred_element_type=jnp.float32) + b_ref[...]
    h = jnp.maximum(h, 0.0)
    o_ref[...] = h * _norm_from(h0[...], h1[...], SRC_COL)


def _layer2_body(a0, a1, h0, h1, w_ref, b_ref, wf_ref, bf_ref, o_ref):
    agg = (a0[...] + a1[...]) * _norm_from(h0[...], h1[...], DST_COL)
    h = jnp.dot(agg, w_ref[...], preferred_element_type=jnp.float32) + b_ref[...]
    o_ref[...] = (
        jnp.dot(h, wf_ref[...], preferred_element_type=jnp.float32) + bf_ref[...]
    )


def _hist_spec(core):
    off = core * (NP // BR)
    return pl.BlockSpec((BR, D), lambda i, o=off: (o + i, 0))


def _acc_spec(core):
    off = core * (NP // BR)
    return pl.BlockSpec((BR, D), lambda i, o=off: (o + i, 0))


_row_spec = pl.BlockSpec((BR, D), lambda i: (i, 0))


_scale = pl.pallas_call(
    _scale_body,
    grid=(_GRID,),
    in_specs=[_row_spec, _hist_spec(0), _hist_spec(1)],
    out_specs=_row_spec,
    out_shape=jax.ShapeDtypeStruct((NP, D), jnp.float32),
)

_layer1 = pl.pallas_call(
    _layer1_body,
    grid=(_GRID,),
    in_specs=[
        _acc_spec(0), _acc_spec(1),
        _hist_spec(0), _hist_spec(1),
        pl.BlockSpec((D, D), lambda i: (0, 0)),
        pl.BlockSpec((1, D), lambda i: (0, 0)),
    ],
    out_specs=_row_spec,
    out_shape=jax.ShapeDtypeStruct((NP, D), jnp.float32),
)

_layer2 = pl.pallas_call(
    _layer2_body,
    grid=(_GRID,),
    in_specs=[
        _acc_spec(0), _acc_spec(1),
        _hist_spec(0), _hist_spec(1),
        pl.BlockSpec((D, D), lambda i: (0, 0)),
        pl.BlockSpec((1, D), lambda i: (0, 0)),
        pl.BlockSpec((D, 1), lambda i: (0, 0)),
        pl.BlockSpec((1, 1), lambda i: (0, 0)),
    ],
    out_specs=pl.BlockSpec((BR, 1), lambda i: (i, 0)),
    out_shape=jax.ShapeDtypeStruct((NP, 1), jnp.float32),
)


def kernel(x, edge_index, W1, b1, W2, b2, Wfc, bfc):
    src = edge_index[0].astype(jnp.int32)
    dst = edge_index[1].astype(jnp.int32)
    pad = jnp.full((EP - E,), N, dtype=jnp.int32)
    src = jnp.concatenate([src, pad])
    dst = jnp.concatenate([dst, pad])
    xp = jnp.pad(x.astype(jnp.float32), ((0, NP - N), (0, 0)))

    zeros = jnp.zeros((NP, D), jnp.float32)

    hist = _hist(src, dst, zeros)
    h0 = _scale(xp, hist, hist)

    acc1 = _agg(h0, src, dst, zeros)
    h1 = _layer1(acc1, acc1, hist, hist,
                 W1, b1.reshape(1, D))

    acc2 = _agg(h1, src, dst, zeros)
    out = _layer2(acc2, acc2, hist, hist,
                  W2, b2.reshape(1, D), Wfc, bfc.reshape(1, 1))
    return out[:N]


# speedup vs baseline: 3.4661x; 1.2113x over previous
import functools

import jax
import jax.numpy as jnp
from jax import lax
from jax.experimental import pallas as pl
from jax.experimental.pallas import tpu as pltpu
from jax.experimental.pallas import tpu_sc as plsc

N = 10000
NP = 10240
E = 320000
D = 128
NC = 2
NS = 16
NW = NC * NS
EPW = 10240
EP = NW * EPW
CH = 128
NCH = EPW // CH
RPW = NP // NS
HW = 16

_mesh = plsc.VectorSubcoreMesh(
    core_axis_name="c", subcore_axis_name="s", num_cores=NC, num_subcores=NS
)


def _hist_body(src_hbm, dst_hbm, zeros_hbm, out_hbm,
               acc, six, dix, pat_s, pat_d):
    c = lax.axis_index("c")
    s = lax.axis_index("s")
    wid = c * NS + s
    one16 = jnp.ones((16,), jnp.float32)
    zero16 = jnp.zeros((16,), jnp.float32)

    def fill(i, carry):
        pat_s[i, pl.ds(0, 16)] = one16
        pat_d[i, pl.ds(0, 16)] = zero16
        pat_s[i, pl.ds(16, 16)] = zero16
        pat_d[i, pl.ds(16, 16)] = one16
        for j in range(2, 8):
            pat_s[i, pl.ds(j * 16, 16)] = zero16
            pat_d[i, pl.ds(j * 16, 16)] = zero16
        return carry

    lax.fori_loop(0, CH, fill, 0)
    pltpu.sync_copy(zeros_hbm.at[pl.ds(s * RPW, RPW)], acc.at[pl.ds(s * RPW, RPW)])
    plsc.subcore_barrier()

    ebase = wid * EPW

    def body(i, carry):
        base = ebase + i * CH
        pltpu.sync_copy(src_hbm.at[pl.ds(base, CH)], six)
        pltpu.sync_copy(dst_hbm.at[pl.ds(base, CH)], dix)
        pltpu.sync_copy(pat_s, acc.at[six], add=True)
        pltpu.sync_copy(pat_d, acc.at[dix], add=True)
        return carry

    lax.fori_loop(0, NCH, body, 0)
    plsc.subcore_barrier()
    pltpu.sync_copy(acc.at[pl.ds(s * RPW, RPW)],
                    out_hbm.at[pl.ds(c * NP + s * RPW, RPW)])


_hist = pl.kernel(
    _hist_body,
    out_type=jax.ShapeDtypeStruct((2 * NP, D), jnp.float32),
    mesh=_mesh,
    scratch_types=[
        pltpu.VMEM_SHARED((NP, D), jnp.float32),
        pltpu.VMEM((CH,), jnp.int32),
        pltpu.VMEM((CH,), jnp.int32),
        pltpu.VMEM((CH, D), jnp.float32),
        pltpu.VMEM((CH, D), jnp.float32),
    ],
)


def _agg_body(h_hbm, src_hbm, dst_hbm, zeros_hbm, out_hbm,
              acc, six0, six1, dix, rows0, rows1, sem0, sem1):
    c = lax.axis_index("c")
    s = lax.axis_index("s")
    wid = c * NS + s
    pltpu.sync_copy(zeros_hbm.at[pl.ds(s * RPW, RPW)], acc.at[pl.ds(s * RPW, RPW)])

    ebase = wid * EPW
    pltpu.sync_copy(src_hbm.at[pl.ds(ebase, CH)], six0)
    pltpu.async_copy(h_hbm.at[six0], rows0, sem0)
    plsc.subcore_barrier()

    def body(k, carry):
        pltpu.sync_copy(src_hbm.at[pl.ds(ebase + (2 * k + 1) * CH, CH)], six1)
        pltpu.async_copy(h_hbm.at[six1], rows1, sem1)
        pltpu.sync_copy(dst_hbm.at[pl.ds(ebase + 2 * k * CH, CH)], dix)
        pltpu.make_async_copy(h_hbm.at[six0], rows0, sem0).wait()
        pltpu.sync_copy(rows0, acc.at[dix], add=True)

        @pl.when(k < NCH // 2 - 1)
        def _():
            pltpu.sync_copy(src_hbm.at[pl.ds(ebase + (2 * k + 2) * CH, CH)], six0)
            pltpu.async_copy(h_hbm.at[six0], rows0, sem0)

        pltpu.sync_copy(dst_hbm.at[pl.ds(ebase + (2 * k + 1) * CH, CH)], dix)
        pltpu.make_async_copy(h_hbm.at[six1], rows1, sem1).wait()
        pltpu.sync_copy(rows1, acc.at[dix], add=True)
        return carry

    lax.fori_loop(0, NCH // 2, body, 0)
    plsc.subcore_barrier()
    pltpu.sync_copy(acc.at[pl.ds(s * RPW, RPW)],
                    out_hbm.at[pl.ds(c * NP + s * RPW, RPW)])


_agg = pl.kernel(
    _agg_body,
    out_type=jax.ShapeDtypeStruct((2 * NP, D), jnp.float32),
    mesh=_mesh,
    scratch_types=[
        pltpu.VMEM_SHARED((NP, D), jnp.float32),
        pltpu.VMEM((CH,), jnp.int32),
        pltpu.VMEM((CH,), jnp.int32),
        pltpu.VMEM((CH,), jnp.int32),
        pltpu.VMEM((CH, D), jnp.float32),
        pltpu.VMEM((CH, D), jnp.float32),
        pltpu.SemaphoreType.DMA,
        pltpu.SemaphoreType.DMA,
    ],
)


BR = 1280
_GRID = NP // BR


SRC_COL = 0
DST_COL = 16


def _norm_from(h0, h1, col):
    deg = h0[:, col:col + 1] + h1[:, col:col + 1]
    return lax.rsqrt(jnp.maximum(deg, 1.0))


def _scale_body(x_ref, h0, h1, o_ref):
    o_ref[...] = x_ref[...] * _norm_from(h0[...], h1[...], SRC_COL)


def _layer1_body(a0, a1, h0, h1, w_ref, b_ref, o_ref):
    agg = (a0[...] + a1[...]) * _norm_from(h0[...], h1[...], DST_COL)
    h = jnp.dot(agg, w_ref[...], preferred_element_type=jnp.float32) + b_ref[...]
    h = jnp.maximum(h, 0.0)
    o_ref[...] = h * _norm_from(h0[...], h1[...], SRC_COL)


def _layer2_body(a0, a1, h0, h1, w_ref, b_ref, wf_ref, bf_ref, o_ref):
    agg = (a0[...] + a1[...]) * _norm_from(h0[...], h1[...], DST_COL)
    h = jnp.dot(agg, w_ref[...], preferred_element_type=jnp.float32) + b_ref[...]
    o_ref[...] = (
        jnp.dot(h, wf_ref[...], preferred_element_type=jnp.float32) + bf_ref[...]
    )


def _hist_spec(core):
    off = core * (NP // BR)
    return pl.BlockSpec((BR, D), lambda i, o=off: (o + i, 0))


def _acc_spec(core):
    off = core * (NP // BR)
    return pl.BlockSpec((BR, D), lambda i, o=off: (o + i, 0))


_row_spec = pl.BlockSpec((BR, D), lambda i: (i, 0))


_scale = pl.pallas_call(
    _scale_body,
    grid=(_GRID,),
    in_specs=[_row_spec, _hist_spec(0), _hist_spec(1)],
    out_specs=_row_spec,
    out_shape=jax.ShapeDtypeStruct((NP, D), jnp.float32),
)

_layer1 = pl.pallas_call(
    _layer1_body,
    grid=(_GRID,),
    in_specs=[
        _acc_spec(0), _acc_spec(1),
        _hist_spec(0), _hist_spec(1),
        pl.BlockSpec((D, D), lambda i: (0, 0)),
        pl.BlockSpec((1, D), lambda i: (0, 0)),
    ],
    out_specs=_row_spec,
    out_shape=jax.ShapeDtypeStruct((NP, D), jnp.float32),
)

_layer2 = pl.pallas_call(
    _layer2_body,
    grid=(_GRID,),
    in_specs=[
        _acc_spec(0), _acc_spec(1),
        _hist_spec(0), _hist_spec(1),
        pl.BlockSpec((D, D), lambda i: (0, 0)),
        pl.BlockSpec((1, D), lambda i: (0, 0)),
        pl.BlockSpec((D, 1), lambda i: (0, 0)),
        pl.BlockSpec((1, 1), lambda i: (0, 0)),
    ],
    out_specs=pl.BlockSpec((BR, 1), lambda i: (i, 0)),
    out_shape=jax.ShapeDtypeStruct((NP, 1), jnp.float32),
)


def kernel(x, edge_index, W1, b1, W2, b2, Wfc, bfc):
    src = edge_index[0].astype(jnp.int32)
    dst = edge_index[1].astype(jnp.int32)
    pad = jnp.full((EP - E,), N, dtype=jnp.int32)
    src = jnp.concatenate([src, pad])
    dst = jnp.concatenate([dst, pad])
    xp = jnp.pad(x.astype(jnp.float32), ((0, NP - N), (0, 0)))

    zeros = jnp.zeros((NP, D), jnp.float32)

    hist = _hist(src, dst, zeros)
    h0 = _scale(xp, hist, hist)

    acc1 = _agg(h0, src, dst, zeros)
    h1 = _layer1(acc1, acc1, hist, hist,
                 W1, b1.reshape(1, D))

    acc2 = _agg(h1, src, dst, zeros)
    out = _layer2(acc2, acc2, hist, hist,
                  W2, b2.reshape(1, D), Wfc, bfc.reshape(1, 1))
    return out[:N]


# speedup vs baseline: 7.1172x; 2.0534x over previous
import functools

import jax
import jax.numpy as jnp
from jax import lax
from jax.experimental import pallas as pl
from jax.experimental.pallas import tpu as pltpu
from jax.experimental.pallas import tpu_sc as plsc

N = 10000
NP = 10240
E = 320000
D = 128
NC = 2
NS = 16
NW = NC * NS
EPW = 10240
EP = NW * EPW
CH = 128
NCH = EPW // CH
RPW = NP // NS
HW = 16

_mesh = plsc.VectorSubcoreMesh(
    core_axis_name="c", subcore_axis_name="s", num_cores=NC, num_subcores=NS
)


def _hist_body(src_hbm, dst_hbm, zeros_hbm, out_hbm,
               acc, six, dix, pat_s, pat_d):
    c = lax.axis_index("c")
    s = lax.axis_index("s")
    wid = c * NS + s
    one16 = jnp.ones((16,), jnp.float32)
    zero16 = jnp.zeros((16,), jnp.float32)

    def fill(i, carry):
        pat_s[i, pl.ds(0, 16)] = one16
        pat_d[i, pl.ds(0, 16)] = zero16
        pat_s[i, pl.ds(16, 16)] = zero16
        pat_d[i, pl.ds(16, 16)] = one16
        for j in range(2, 8):
            pat_s[i, pl.ds(j * 16, 16)] = zero16
            pat_d[i, pl.ds(j * 16, 16)] = zero16
        return carry

    lax.fori_loop(0, CH, fill, 0)
    pltpu.sync_copy(zeros_hbm.at[pl.ds(s * RPW, RPW)], acc.at[pl.ds(s * RPW, RPW)])
    plsc.subcore_barrier()

    ebase = wid * EPW

    def body(i, carry):
        base = ebase + i * CH
        pltpu.sync_copy(src_hbm.at[pl.ds(base, CH)], six)
        pltpu.sync_copy(dst_hbm.at[pl.ds(base, CH)], dix)
        pltpu.sync_copy(pat_s, acc.at[six], add=True)
        pltpu.sync_copy(pat_d, acc.at[dix], add=True)
        return carry

    lax.fori_loop(0, NCH, body, 0)
    plsc.subcore_barrier()
    pltpu.sync_copy(acc.at[pl.ds(s * RPW, RPW)],
                    out_hbm.at[pl.ds(c * NP + s * RPW, RPW)])


_hist = pl.kernel(
    _hist_body,
    out_type=jax.ShapeDtypeStruct((2 * NP, D), jnp.float32),
    mesh=_mesh,
    scratch_types=[
        pltpu.VMEM_SHARED((NP, D), jnp.float32),
        pltpu.VMEM((CH,), jnp.int32),
        pltpu.VMEM((CH,), jnp.int32),
        pltpu.VMEM((CH, D), jnp.float32),
        pltpu.VMEM((CH, D), jnp.float32),
    ],
)


def _agg_body(h_hbm, src_hbm, dst_hbm, zeros_hbm, out_hbm,
              acc, six0, six1, dix, rows0, rows1, sem0, sem1):
    c = lax.axis_index("c")
    s = lax.axis_index("s")
    wid = c * NS + s
    pltpu.sync_copy(zeros_hbm.at[pl.ds(s * RPW, RPW)], acc.at[pl.ds(s * RPW, RPW)])

    ebase = wid * EPW
    pltpu.sync_copy(src_hbm.at[pl.ds(ebase, CH)], six0)
    pltpu.async_copy(h_hbm.at[six0], rows0, sem0)
    plsc.subcore_barrier()

    def body(k, carry):
        pltpu.sync_copy(src_hbm.at[pl.ds(ebase + (2 * k + 1) * CH, CH)], six1)
        pltpu.async_copy(h_hbm.at[six1], rows1, sem1)
        pltpu.sync_copy(dst_hbm.at[pl.ds(ebase + 2 * k * CH, CH)], dix)
        pltpu.make_async_copy(h_hbm.at[six0], rows0, sem0).wait()
        pltpu.sync_copy(rows0, acc.at[dix], add=True)

        @pl.when(k < NCH // 2 - 1)
        def _():
            pltpu.sync_copy(src_hbm.at[pl.ds(ebase + (2 * k + 2) * CH, CH)], six0)
            pltpu.async_copy(h_hbm.at[six0], rows0, sem0)

        pltpu.sync_copy(dst_hbm.at[pl.ds(ebase + (2 * k + 1) * CH, CH)], dix)
        pltpu.make_async_copy(h_hbm.at[six1], rows1, sem1).wait()
        pltpu.sync_copy(rows1, acc.at[dix], add=True)
        return carry

    lax.fori_loop(0, NCH // 2, body, 0)
    plsc.subcore_barrier()
    pltpu.sync_copy(acc.at[pl.ds(s * RPW, RPW)],
                    out_hbm.at[pl.ds(c * NP + s * RPW, RPW)])


_agg = pl.kernel(
    _agg_body,
    out_type=jax.ShapeDtypeStruct((2 * NP, D), jnp.float32),
    mesh=_mesh,
    scratch_types=[
        pltpu.VMEM_SHARED((NP, D), jnp.float32),
        pltpu.VMEM((CH,), jnp.int32),
        pltpu.VMEM((CH,), jnp.int32),
        pltpu.VMEM((CH,), jnp.int32),
        pltpu.VMEM((CH, D), jnp.float32),
        pltpu.VMEM((CH, D), jnp.float32),
        pltpu.SemaphoreType.DMA,
        pltpu.SemaphoreType.DMA,
    ],
)


BR = 1280
_GRID = NP // BR


SRC_COL = 0
DST_COL = 16


def _norm_from(h0, h1, col):
    deg = h0[:, col:col + 1] + h1[:, col:col + 1]
    return lax.rsqrt(jnp.maximum(deg, 1.0))


def _scale_body(x_ref, h0, h1, o_ref):
    o_ref[...] = x_ref[...] * _norm_from(h0[...], h1[...], SRC_COL)


def _layer1_body(a0, a1, h0, h1, w_ref, b_ref, o_ref):
    agg = (a0[...] + a1[...]) * _norm_from(h0[...], h1[...], DST_COL)
    h = jnp.dot(agg, w_ref[...], preferred_element_type=jnp.float32) + b_ref[...]
    h = jnp.maximum(h, 0.0)
    o_ref[...] = h * _norm_from(h0[...], h1[...], SRC_COL)


def _layer2_body(a0, a1, h0, h1, w_ref, b_ref, wf_ref, bf_ref, o_ref):
    agg = (a0[...] + a1[...]) * _norm_from(h0[...], h1[...], DST_COL)
    h = jnp.dot(agg, w_ref[...], preferred_element_type=jnp.float32) + b_ref[...]
    o_ref[...] = (
        jnp.dot(h, wf_ref[...], preferred_element_type=jnp.float32) + bf_ref[...]
    )


def _hist_spec(core):
    off = core * (NP // BR)
    return pl.BlockSpec((BR, D), lambda i, o=off: (o + i, 0))


def _acc_spec(core):
    off = core * (NP // BR)
    return pl.BlockSpec((BR, D), lambda i, o=off: (o + i, 0))


_row_spec = pl.BlockSpec((BR, D), lambda i: (i, 0))


_scale = pl.pallas_call(
    _scale_body,
    grid=(_GRID,),
    in_specs=[_row_spec, _hist_spec(0), _hist_spec(1)],
    out_specs=_row_spec,
    out_shape=jax.ShapeDtypeStruct((NP, D), jnp.float32),
)

_layer1 = pl.pallas_call(
    _layer1_body,
    grid=(_GRID,),
    in_specs=[
        _acc_spec(0), _acc_spec(1),
        _hist_spec(0), _hist_spec(1),
        pl.BlockSpec((D, D), lambda i: (0, 0)),
        pl.BlockSpec((1, D), lambda i: (0, 0)),
    ],
    out_specs=_row_spec,
    out_shape=jax.ShapeDtypeStruct((NP, D), jnp.float32),
)

_layer2 = pl.pallas_call(
    _layer2_body,
    grid=(_GRID,),
    in_specs=[
        _acc_spec(0), _acc_spec(1),
        _hist_spec(0), _hist_spec(1),
        pl.BlockSpec((D, D), lambda i: (0, 0)),
        pl.BlockSpec((1, D), lambda i: (0, 0)),
        pl.BlockSpec((D, 1), lambda i: (0, 0)),
        pl.BlockSpec((1, 1), lambda i: (0, 0)),
    ],
    out_specs=pl.BlockSpec((BR, 1), lambda i: (i, 0)),
    out_shape=jax.ShapeDtypeStruct((NP, 1), jnp.float32),
)


def kernel(x, edge_index, W1, b1, W2, b2, Wfc, bfc):
    src = edge_index[0].astype(jnp.int32)
    dst = edge_index[1].astype(jnp.int32)
    pad = N + jnp.arange(EP - E, dtype=jnp.int32) % (NP - N)
    src = jnp.concatenate([src, pad])
    dst = jnp.concatenate([dst, pad])
    xp = jnp.pad(x.astype(jnp.float32), ((0, NP - N), (0, 0)))

    zeros = jnp.zeros((NP, D), jnp.float32)

    hist = _hist(src, dst, zeros)
    h0 = _scale(xp, hist, hist)

    acc1 = _agg(h0, src, dst, zeros)
    h1 = _layer1(acc1, acc1, hist, hist,
                 W1, b1.reshape(1, D))

    acc2 = _agg(h1, src, dst, zeros)
    out = _layer2(acc2, acc2, hist, hist,
                  W2, b2.reshape(1, D), Wfc, bfc.reshape(1, 1))
    return out[:N]


# speedup vs baseline: 9.6308x; 1.3532x over previous
import functools

import jax
import jax.numpy as jnp
from jax import lax
from jax.experimental import pallas as pl
from jax.experimental.pallas import tpu as pltpu
from jax.experimental.pallas import tpu_sc as plsc

N = 10000
NP = 10240
E = 320000
D = 128
NC = 2
NS = 16
NW = NC * NS
EPW = 10240
EP = NW * EPW
CH = 128
NCH = EPW // CH
RPW = NP // NS
HW = 16

_mesh = plsc.VectorSubcoreMesh(
    core_axis_name="c", subcore_axis_name="s", num_cores=NC, num_subcores=NS
)


def _hist_body(src_hbm, dst_hbm, zeros_hbm, out_hbm,
               acc, six, dix, pat_s, pat_d, isem):
    c = lax.axis_index("c")
    s = lax.axis_index("s")
    wid = c * NS + s
    one16 = jnp.ones((16,), jnp.float32)
    zero16 = jnp.zeros((16,), jnp.float32)

    def fill(i, carry):
        pat_s[i, pl.ds(0, 16)] = one16
        pat_d[i, pl.ds(0, 16)] = zero16
        pat_s[i, pl.ds(16, 16)] = zero16
        pat_d[i, pl.ds(16, 16)] = one16
        for j in range(2, 8):
            pat_s[i, pl.ds(j * 16, 16)] = zero16
            pat_d[i, pl.ds(j * 16, 16)] = zero16
        return carry

    lax.fori_loop(0, CH, fill, 0)

    ebase = wid * EPW

    def fire_idx(j, slot):
        pltpu.async_copy(src_hbm.at[pl.ds(ebase + j * CH, CH)], six[slot], isem[slot])
        pltpu.async_copy(dst_hbm.at[pl.ds(ebase + j * CH, CH)], dix[slot], isem[slot])

    def wait_idx(slot):
        pltpu.make_async_copy(src_hbm.at[pl.ds(ebase, CH)], six[slot], isem[slot]).wait()
        pltpu.make_async_copy(dst_hbm.at[pl.ds(ebase, CH)], dix[slot], isem[slot]).wait()

    fire_idx(0, 0)
    pltpu.sync_copy(zeros_hbm.at[pl.ds(s * RPW, RPW)], acc.at[pl.ds(s * RPW, RPW)])
    plsc.subcore_barrier()

    def body(k, carry):
        for p in range(2):
            j = 2 * k + p

            @pl.when(j < NCH - 1)
            def _():
                fire_idx(j + 1, (p + 1) % 2)

            wait_idx(p)
            pltpu.sync_copy(pat_s, acc.at[six[p]], add=True)
            pltpu.sync_copy(pat_d, acc.at[dix[p]], add=True)
        return carry

    lax.fori_loop(0, NCH // 2, body, 0)
    plsc.subcore_barrier()
    pltpu.sync_copy(acc.at[pl.ds(s * RPW, RPW)],
                    out_hbm.at[pl.ds(c * NP + s * RPW, RPW)])


_hist = pl.kernel(
    _hist_body,
    out_type=jax.ShapeDtypeStruct((2 * NP, D), jnp.float32),
    mesh=_mesh,
    scratch_types=[
        pltpu.VMEM_SHARED((NP, D), jnp.float32),
        [pltpu.VMEM((CH,), jnp.int32)] * 2,
        [pltpu.VMEM((CH,), jnp.int32)] * 2,
        pltpu.VMEM((CH, D), jnp.float32),
        pltpu.VMEM((CH, D), jnp.float32),
        [pltpu.SemaphoreType.DMA] * 2,
    ],
)


def _agg_body(h_hbm, src_hbm, dst_hbm, zeros_hbm, out_hbm,
              acc, six, dix, rows, isem, rsem):
    c = lax.axis_index("c")
    s = lax.axis_index("s")
    wid = c * NS + s
    pltpu.sync_copy(zeros_hbm.at[pl.ds(s * RPW, RPW)], acc.at[pl.ds(s * RPW, RPW)])

    ebase = wid * EPW

    def fire_idx(j, slot):
        pltpu.async_copy(src_hbm.at[pl.ds(ebase + j * CH, CH)], six[slot], isem[slot])
        pltpu.async_copy(dst_hbm.at[pl.ds(ebase + j * CH, CH)], dix[slot], isem[slot])

    def wait_idx(slot):
        pltpu.make_async_copy(src_hbm.at[pl.ds(ebase, CH)], six[slot], isem[slot]).wait()
        pltpu.make_async_copy(dst_hbm.at[pl.ds(ebase, CH)], dix[slot], isem[slot]).wait()

    fire_idx(0, 0)
    fire_idx(1, 1)
    wait_idx(0)
    pltpu.async_copy(h_hbm.at[six[0]], rows[0], rsem[0])
    plsc.subcore_barrier()

    def body(t, carry):
        for p in range(4):
            k = 4 * t + p
            s1, s2 = (p + 1) % 4, (p + 2) % 4
            r0, r1 = p % 2, (p + 1) % 2

            @pl.when(k < NCH - 1)
            def _():
                wait_idx(s1)
                pltpu.async_copy(h_hbm.at[six[s1]], rows[r1], rsem[r1])

            pltpu.make_async_copy(h_hbm.at[six[p]], rows[r0], rsem[r0]).wait()

            @pl.when(k < NCH - 2)
            def _():
                fire_idx(k + 2, s2)

            pltpu.sync_copy(rows[r0], acc.at[dix[p]], add=True)
        return carry

    lax.fori_loop(0, NCH // 4, body, 0)
    plsc.subcore_barrier()
    pltpu.sync_copy(acc.at[pl.ds(s * RPW, RPW)],
                    out_hbm.at[pl.ds(c * NP + s * RPW, RPW)])


_agg = pl.kernel(
    _agg_body,
    out_type=jax.ShapeDtypeStruct((2 * NP, D), jnp.float32),
    mesh=_mesh,
    scratch_types=[
        pltpu.VMEM_SHARED((NP, D), jnp.float32),
        [pltpu.VMEM((CH,), jnp.int32)] * 4,
        [pltpu.VMEM((CH,), jnp.int32)] * 4,
        [pltpu.VMEM((CH, D), jnp.float32)] * 2,
        [pltpu.SemaphoreType.DMA] * 4,
        [pltpu.SemaphoreType.DMA] * 2,
    ],
)


BR = 1280
_GRID = NP // BR


SRC_COL = 0
DST_COL = 16


def _norm_from(h0, h1, col):
    deg = h0[:, col:col + 1] + h1[:, col:col + 1]
    return lax.rsqrt(jnp.maximum(deg, 1.0))


def _scale_body(x_ref, h0, h1, o_ref):
    o_ref[...] = x_ref[...] * _norm_from(h0[...], h1[...], SRC_COL)


def _layer1_body(a0, a1, h0, h1, w_ref, b_ref, o_ref):
    agg = (a0[...] + a1[...]) * _norm_from(h0[...], h1[...], DST_COL)
    h = jnp.dot(agg, w_ref[...], preferred_element_type=jnp.float32) + b_ref[...]
    h = jnp.maximum(h, 0.0)
    o_ref[...] = h * _norm_from(h0[...], h1[...], SRC_COL)


def _layer2_body(a0, a1, h0, h1, w_ref, b_ref, wf_ref, bf_ref, o_ref):
    agg = (a0[...] + a1[...]) * _norm_from(h0[...], h1[...], DST_COL)
    h = jnp.dot(agg, w_ref[...], preferred_element_type=jnp.float32) + b_ref[...]
    o_ref[...] = (
        jnp.dot(h, wf_ref[...], preferred_element_type=jnp.float32) + bf_ref[...]
    )


def _hist_spec(core):
    off = core * (NP // BR)
    return pl.BlockSpec((BR, D), lambda i, o=off: (o + i, 0))


def _acc_spec(core):
    off = core * (NP // BR)
    return pl.BlockSpec((BR, D), lambda i, o=off: (o + i, 0))


_row_spec = pl.BlockSpec((BR, D), lambda i: (i, 0))


_scale = pl.pallas_call(
    _scale_body,
    grid=(_GRID,),
    in_specs=[_row_spec, _hist_spec(0), _hist_spec(1)],
    out_specs=_row_spec,
    out_shape=jax.ShapeDtypeStruct((NP, D), jnp.float32),
)

_layer1 = pl.pallas_call(
    _layer1_body,
    grid=(_GRID,),
    in_specs=[
        _acc_spec(0), _acc_spec(1),
        _hist_spec(0), _hist_spec(1),
        pl.BlockSpec((D, D), lambda i: (0, 0)),
        pl.BlockSpec((1, D), lambda i: (0, 0)),
    ],
    out_specs=_row_spec,
    out_shape=jax.ShapeDtypeStruct((NP, D), jnp.float32),
)

_layer2 = pl.pallas_call(
    _layer2_body,
    grid=(_GRID,),
    in_specs=[
        _acc_spec(0), _acc_spec(1),
        _hist_spec(0), _hist_spec(1),
        pl.BlockSpec((D, D), lambda i: (0, 0)),
        pl.BlockSpec((1, D), lambda i: (0, 0)),
        pl.BlockSpec((D, 1), lambda i: (0, 0)),
        pl.BlockSpec((1, 1), lambda i: (0, 0)),
    ],
    out_specs=pl.BlockSpec((BR, 1), lambda i: (i, 0)),
    out_shape=jax.ShapeDtypeStruct((NP, 1), jnp.float32),
)


def kernel(x, edge_index, W1, b1, W2, b2, Wfc, bfc):
    src = edge_index[0].astype(jnp.int32)
    dst = edge_index[1].astype(jnp.int32)
    pad = N + jnp.arange(EP - E, dtype=jnp.int32) % (NP - N)
    src = jnp.concatenate([src, pad])
    dst = jnp.concatenate([dst, pad])
    xp = jnp.pad(x.astype(jnp.float32), ((0, NP - N), (0, 0)))

    zeros = jnp.zeros((NP, D), jnp.float32)

    hist = _hist(src, dst, zeros)
    h0 = _scale(xp, hist, hist)

    acc1 = _agg(h0, src, dst, zeros)
    h1 = _layer1(acc1, acc1, hist, hist,
                 W1, b1.reshape(1, D))

    acc2 = _agg(h1, src, dst, zeros)
    out = _layer2(acc2, acc2, hist, hist,
                  W2, b2.reshape(1, D), Wfc, bfc.reshape(1, 1))
    return out[:N]
